# baseline (speedup 1.0000x reference)
import functools

import jax
import jax.numpy as jnp
from jax import lax
from jax.experimental import pallas as pl
from jax.experimental.pallas import tpu as pltpu
from jax.experimental.pallas import tpu_sc as plsc

N = 10000
NPAD = 10240
E_IN = 160000
ETOT = E_IN + N
EPAD = 172032
NC, NS = 2, 16
DCHUNK = EPAD // NS
TCHUNK = EPAD // (NC * NS)
TBLK = TCHUNK // 128
NSTRIPE = NPAD // NS


def _proj_body(x_ref, w_ref, af_ref, df_ref, xps_ref, asrc_ref, adst_ref):
    s = pl.program_id(1)
    xp = jnp.dot(x_ref[...], w_ref[...], preferred_element_type=jnp.float32)
    xps_ref[0] = xp
    pa = jnp.sum(xp * af_ref[...], axis=1)[None, None, :]
    pd = jnp.sum(xp * df_ref[...], axis=1)[None, None, :]

    @pl.when(s % 2 == 0)
    def _():
        asrc_ref[...] = pa
        adst_ref[...] = pd

    @pl.when(s % 2 == 1)
    def _():
        asrc_ref[...] += pa
        adst_ref[...] += pd


def _project_l1(xpad, W1, asf, adf):
    return pl.pallas_call(
        _proj_body,
        grid=(NPAD // 512, 16),
        in_specs=[
            pl.BlockSpec((512, 256), lambda i, s: (i, 0)),
            pl.BlockSpec((256, 128), lambda i, s: (0, s)),
            pl.BlockSpec((1, 128), lambda i, s: (0, s)),
            pl.BlockSpec((1, 128), lambda i, s: (0, s)),
        ],
        out_specs=[
            pl.BlockSpec((1, 512, 128), lambda i, s: (s, i, 0)),
            pl.BlockSpec((1, 1, 512), lambda i, s: (s // 2, 0, i)),
            pl.BlockSpec((1, 1, 512), lambda i, s: (s // 2, 0, i)),
        ],
        out_shape=[
            jax.ShapeDtypeStruct((16, NPAD, 128), jnp.float32),
            jax.ShapeDtypeStruct((8, 1, NPAD), jnp.float32),
            jax.ShapeDtypeStruct((8, 1, NPAD), jnp.float32),
        ],
    )(xpad, W1, asf, adf)


def _mid_body(p_ref, b_ref, w_ref, a2_ref, d2_ref, xp2_ref, as2_ref, ad2_ref):
    k = pl.program_id(1)
    hblk = p_ref[0, 0] + p_ref[1, 0] + b_ref[0]
    hblk = jnp.where(hblk > 0.0, hblk, jnp.exp(jnp.minimum(hblk, 0.0)) - 1.0)
    acc = jnp.dot(hblk, w_ref[...], preferred_element_type=jnp.float32)

    @pl.when(k == 0)
    def _():
        xp2_ref[...] = acc

    @pl.when(k > 0)
    def _():
        xp2_ref[...] += acc

    @pl.when(k == 15)
    def _():
        xp2 = xp2_ref[...]
        as2_ref[...] = jnp.sum(xp2 * a2_ref[...], axis=1)[None, :]
        ad2_ref[...] = jnp.sum(xp2 * d2_ref[...], axis=1)[None, :]


def _mid(outp1, b1r, W2, as2f, ad2f):
    return pl.pallas_call(
        _mid_body,
        grid=(NPAD // 512, 16),
        in_specs=[
            pl.BlockSpec((2, 1, 512, 128), lambda i, k: (0, k, i, 0)),
            pl.BlockSpec((1, 1, 128), lambda i, k: (k, 0, 0)),
            pl.BlockSpec((128, 256), lambda i, k: (k, 0)),
            pl.BlockSpec((1, 256), lambda i, k: (0, 0)),
            pl.BlockSpec((1, 256), lambda i, k: (0, 0)),
        ],
        out_specs=[
            pl.BlockSpec((512, 256), lambda i, k: (i, 0)),
            pl.BlockSpec((1, 512), lambda i, k: (0, i)),
            pl.BlockSpec((1, 512), lambda i, k: (0, i)),
        ],
        out_shape=[
            jax.ShapeDtypeStruct((NPAD, 256), jnp.float32),
            jax.ShapeDtypeStruct((1, NPAD), jnp.float32),
            jax.ShapeDtypeStruct((1, NPAD), jnp.float32),
        ],
    )(outp1, b1r, W2, as2f, ad2f)


def _final_body(p_ref, b_ref, out_ref):
    o0 = p_ref[0, 0] + p_ref[1, 0]
    o1 = p_ref[0, 1] + p_ref[1, 1]
    out_ref[...] = jnp.concatenate([o0, o1], axis=1) + b_ref[...]


def _final(outp2, b2r):
    return pl.pallas_call(
        _final_body,
        grid=(NPAD // 512,),
        in_specs=[
            pl.BlockSpec((2, 2, 512, 128), lambda i: (0, 0, i, 0)),
            pl.BlockSpec((1, 256), lambda i: (0, 0)),
        ],
        out_specs=pl.BlockSpec((512, 256), lambda i: (i, 0)),
        out_shape=jax.ShapeDtypeStruct((NPAD, 256), jnp.float32),
    )(outp2, b2r)


def _sc_mesh():
    return plsc.VectorSubcoreMesh(
        core_axis_name="c", subcore_axis_name="s", num_cores=NC, num_subcores=NS
    )


def _make_coef_kernel(H):
    @functools.partial(
        pl.kernel,
        out_type=jax.ShapeDtypeStruct((H, EPAD), jnp.float32),
        mesh=_sc_mesh(),
        compiler_params=pltpu.CompilerParams(needs_layout_passes=False),
        scratch_types=[
            pltpu.VMEM((DCHUNK,), jnp.int32),
            pltpu.VMEM((DCHUNK,), jnp.int32),
            pltpu.VMEM((TCHUNK,), jnp.int32),
            pltpu.VMEM((TCHUNK,), jnp.int32),
            pltpu.VMEM((NPAD,), jnp.float32),
            pltpu.VMEM((NPAD,), jnp.float32),
            pltpu.VMEM((128, 128), jnp.float32),
            pltpu.VMEM((128, 128), jnp.float32),
            pltpu.VMEM((TCHUNK,), jnp.float32),
            pltpu.VMEM((1, 128), jnp.int32),
            pltpu.VMEM_SHARED((128, 128), jnp.float32),
        ],
    )
    def coef_kernel(src_hbm, dst_hbm, as_hbm, ad_hbm, z_hbm, idx_hbm, coef_hbm,
                    src_d, dst_d, src_c, dst_c, as_t, ad_t, dacc, den_t,
                    coef_buf, idxv, denom_sh):
        c = lax.axis_index("c")
        s = lax.axis_index("s")
        doff = pl.multiple_of(s * DCHUNK, 8)
        woff = pl.multiple_of((c * NS + s) * TCHUNK, 8)
        pltpu.sync_copy(src_hbm.at[pl.ds(doff, DCHUNK)], src_d)
        pltpu.sync_copy(dst_hbm.at[pl.ds(doff, DCHUNK)], dst_d)
        pltpu.sync_copy(src_hbm.at[pl.ds(woff, TCHUNK)], src_c)
        pltpu.sync_copy(dst_hbm.at[pl.ds(woff, TCHUNK)], dst_c)
        pltpu.sync_copy(idx_hbm, idxv)

        for h in range(H):
            pltpu.sync_copy(as_hbm.at[h], as_t)
            pltpu.sync_copy(ad_hbm.at[h], ad_t)
            pltpu.sync_copy(z_hbm, dacc)

            @pl.when(s == 0)
            def _():
                pltpu.sync_copy(z_hbm, denom_sh)

            plsc.subcore_barrier()

            @pl.loop(0, DCHUNK // 16)
            def _(e):
                sl = pl.ds(e * 16, 16)
                sv = src_d[sl]
                dv = dst_d[sl]
                a = plsc.load_gather(as_t, [sv]) + plsc.load_gather(ad_t, [dv])
                a = jnp.where(a >= 0.0, a, 0.2 * a)
                plsc.addupdate_scatter(dacc, [dv >> 7, dv & 127], jnp.exp(a))

            pltpu.sync_copy(dacc, denom_sh.at[idxv.at[0]], add=True)
            plsc.subcore_barrier()
            pltpu.sync_copy(denom_sh, den_t)

            @pl.loop(0, TCHUNK // 16)
            def _(e):
                sl = pl.ds(e * 16, 16)
                sv = src_c[sl]
                dv = dst_c[sl]
                a = plsc.load_gather(as_t, [sv]) + plsc.load_gather(ad_t, [dv])
                a = jnp.where(a >= 0.0, a, 0.2 * a)
                dn = plsc.load_gather(den_t, [dv >> 7, dv & 127])
                coef_buf[sl] = jnp.exp(a) / (dn + 1e-16)

            pltpu.sync_copy(coef_buf, coef_hbm.at[h, pl.ds(woff, TCHUNK)])
            plsc.subcore_barrier()

    return coef_kernel


def _make_agg_kernel(nslab):
    @functools.partial(
        pl.kernel,
        out_type=jax.ShapeDtypeStruct((NC, nslab, NPAD, 128), jnp.float32),
        mesh=_sc_mesh(),
        compiler_params=pltpu.CompilerParams(needs_layout_passes=False),
        scratch_types=[
            pltpu.VMEM((TBLK, 128), jnp.int32),
            pltpu.VMEM((TBLK, 128), jnp.int32),
            pltpu.VMEM((128, 128), jnp.float32),
            pltpu.VMEM((128, 128), jnp.float32),
            pltpu.VMEM((8, 128), jnp.float32),
            pltpu.VMEM_SHARED((NPAD, 128), jnp.float32),
            pltpu.SemaphoreType.DMA,
            pltpu.SemaphoreType.DMA,
            pltpu.SemaphoreType.DMA,
            pltpu.SemaphoreType.DMA,
        ],
    )
    def agg_kernel(src2_hbm, dst2_hbm, coef_hbm, xps_hbm, zrow_hbm, outp_hbm,
                   src2, dst2, rows0, rows1, cbuf, acc,
                   gsem0, gsem1, ssem0, ssem1):
        c = lax.axis_index("c")
        s = lax.axis_index("s")
        wid = c * NS + s
        stripe = pl.multiple_of(s * NSTRIPE, 8)
        pltpu.sync_copy(src2_hbm.at[wid], src2)
        pltpu.sync_copy(dst2_hbm.at[wid], dst2)

        rbufs = (rows0, rows1)
        gsems = (gsem0, gsem1)
        ssems = (ssem0, ssem1)

        def _scale(rows_ref, crow):
            @pl.loop(0, 8)
            def _(g):
                cv = cbuf[crow, pl.ds(g * 16, 16)]
                for e in range(16):
                    sc = cv[e]
                    erow = g * 16 + e
                    for r in range(8):
                        rows_ref[erow, pl.ds(r * 16, 16)] *= sc

        @pl.loop(0, nslab)
        def _(slab):
            head = slab >> 1
            xs = xps_hbm.at[slab]

            @pl.loop(0, NSTRIPE // 128)
            def _(j):
                pltpu.sync_copy(zrow_hbm, acc.at[pl.ds(stripe + j * 128, 128)])

            plsc.subcore_barrier()

            for b in range(2):
                pltpu.async_copy(xs.at[src2.at[b]], rbufs[b], gsems[b])
                pltpu.sync_copy(coef_hbm.at[head, wid, b], cbuf.at[b])

            @pl.loop(0, TBLK // 2)
            def _(pair):
                j0 = 2 * pair
                for b in range(2):
                    jb = j0 + b
                    pltpu.make_async_copy(
                        xs.at[src2.at[jb]], rbufs[b], gsems[b]).wait()
                    _scale(rbufs[b], b)
                    pltpu.async_copy(
                        rbufs[b], acc.at[dst2.at[jb]], ssems[b], add=True)

                @pl.when(pair < TBLK // 2 - 1)
                def _():
                    for b in range(2):
                        jb = j0 + b
                        pltpu.make_async_copy(
                            rbufs[b], acc.at[dst2.at[jb]], ssems[b]).wait()
                        pltpu.sync_copy(
                            coef_hbm.at[head, wid, jb + 2], cbuf.at[b])
                        pltpu.async_copy(
                            xs.at[src2.at[jb + 2]], rbufs[b], gsems[b])

            for b in range(2):
                pltpu.make_async_copy(
                    rbufs[b], acc.at[dst2.at[TBLK - 2 + b]], ssems[b]).wait()

            plsc.subcore_barrier()

            @pl.loop(0, NSTRIPE // 128)
            def _(j):
                roff = pl.multiple_of(stripe + j * 128, 8)
                pltpu.sync_copy(acc.at[pl.ds(roff, 128)], rows0)
                pltpu.sync_copy(rows0, outp_hbm.at[c, slab, pl.ds(roff, 128)])

    return agg_kernel


_coef8 = _make_coef_kernel(8)
_coef1 = _make_coef_kernel(1)
_agg16 = _make_agg_kernel(16)
_agg2 = _make_agg_kernel(2)


def kernel(x, edge_index, W1, as1, ad1, b1, W2, as2, ad2, b2):
    loop = jnp.arange(N, dtype=edge_index.dtype)
    ei = jnp.concatenate([edge_index, jnp.stack([loop, loop])], axis=1)
    padlen = EPAD - ETOT
    src_p = jnp.concatenate(
        [ei[0], jnp.zeros((padlen,), jnp.int32)]).astype(jnp.int32)
    dst_p = jnp.concatenate(
        [ei[1], jnp.full((padlen,), N, jnp.int32)]).astype(jnp.int32)
    wids = jnp.arange(NC * NS)
    perm = (wids % NS) * NC + wids // NS
    src2d = src_p.reshape(NC * NS, TBLK, 128)[perm]
    dst2d = dst_p.reshape(NC * NS, TBLK, 128)[perm]

    xpad = jnp.pad(x, ((0, NPAD - N), (0, 0)))
    zrow = jnp.zeros((128, 128), jnp.float32)
    idx128 = jnp.arange(128, dtype=jnp.int32).reshape(1, 128)

    xps1, asrc1, adst1 = _project_l1(
        xpad, W1, as1.reshape(1, -1), ad1.reshape(1, -1))
    coef1 = _coef8(src_p, dst_p, asrc1.reshape(8, NPAD),
                   adst1.reshape(8, NPAD), zrow, idx128)
    outp1 = _agg16(src2d, dst2d,
                   coef1.reshape(8, NC * NS, TBLK, 128)[:, perm], xps1, zrow)

    xp2, asrc2, adst2 = _mid(outp1, b1.reshape(16, 1, 128), W2,
                             as2.reshape(1, -1), ad2.reshape(1, -1))
    xps2 = jnp.stack([xp2[:, :128], xp2[:, 128:]], axis=0)
    coef2 = _coef1(src_p, dst_p, asrc2, adst2, zrow, idx128)
    outp2 = _agg2(src2d, dst2d,
                  coef2.reshape(1, NC * NS, TBLK, 128)[:, perm], xps2, zrow)

    out = _final(outp2, b2.reshape(1, -1))
    return out[:N]


# speedup vs baseline: 1.0121x; 1.0121x over previous
import functools

import jax
import jax.numpy as jnp
from jax import lax
from jax.experimental import pallas as pl
from jax.experimental.pallas import tpu as pltpu
from jax.experimental.pallas import tpu_sc as plsc

N = 10000
NPAD = 10240
E_IN = 160000
ETOT = E_IN + N
EPAD = 172032
NC, NS = 2, 16
DCHUNK = EPAD // NS
TCHUNK = EPAD // (NC * NS)
TBLK = TCHUNK // 128
NSTRIPE = NPAD // NS


def _proj_body(x_ref, w_ref, af_ref, df_ref, xps_ref, asrc_ref, adst_ref):
    s = pl.program_id(1)
    xp = jnp.dot(x_ref[...], w_ref[...], preferred_element_type=jnp.float32)
    xps_ref[0] = xp
    pa = jnp.sum(xp * af_ref[...], axis=1)[None, None, :]
    pd = jnp.sum(xp * df_ref[...], axis=1)[None, None, :]

    @pl.when(s % 2 == 0)
    def _():
        asrc_ref[...] = pa
        adst_ref[...] = pd

    @pl.when(s % 2 == 1)
    def _():
        asrc_ref[...] += pa
        adst_ref[...] += pd


def _project_l1(xpad, W1, asf, adf):
    return pl.pallas_call(
        _proj_body,
        grid=(NPAD // 512, 16),
        in_specs=[
            pl.BlockSpec((512, 256), lambda i, s: (i, 0)),
            pl.BlockSpec((256, 128), lambda i, s: (0, s)),
            pl.BlockSpec((1, 128), lambda i, s: (0, s)),
            pl.BlockSpec((1, 128), lambda i, s: (0, s)),
        ],
        out_specs=[
            pl.BlockSpec((1, 512, 128), lambda i, s: (s, i, 0)),
            pl.BlockSpec((1, 1, 512), lambda i, s: (s // 2, 0, i)),
            pl.BlockSpec((1, 1, 512), lambda i, s: (s // 2, 0, i)),
        ],
        out_shape=[
            jax.ShapeDtypeStruct((16, NPAD, 128), jnp.float32),
            jax.ShapeDtypeStruct((8, 1, NPAD), jnp.float32),
            jax.ShapeDtypeStruct((8, 1, NPAD), jnp.float32),
        ],
    )(xpad, W1, asf, adf)


def _mid_body(p_ref, b_ref, w_ref, a2_ref, d2_ref, xp2_ref, as2_ref, ad2_ref):
    k = pl.program_id(1)
    hblk = p_ref[0, 0] + p_ref[1, 0] + b_ref[0]
    hblk = jnp.where(hblk > 0.0, hblk, jnp.exp(jnp.minimum(hblk, 0.0)) - 1.0)
    acc = jnp.dot(hblk, w_ref[...], preferred_element_type=jnp.float32)

    @pl.when(k == 0)
    def _():
        xp2_ref[...] = acc

    @pl.when(k > 0)
    def _():
        xp2_ref[...] += acc

    @pl.when(k == 15)
    def _():
        xp2 = xp2_ref[...]
        as2_ref[...] = jnp.sum(xp2 * a2_ref[...], axis=1)[None, :]
        ad2_ref[...] = jnp.sum(xp2 * d2_ref[...], axis=1)[None, :]


def _mid(outp1, b1r, W2, as2f, ad2f):
    return pl.pallas_call(
        _mid_body,
        grid=(NPAD // 512, 16),
        in_specs=[
            pl.BlockSpec((2, 1, 512, 128), lambda i, k: (0, k, i, 0)),
            pl.BlockSpec((1, 1, 128), lambda i, k: (k, 0, 0)),
            pl.BlockSpec((128, 256), lambda i, k: (k, 0)),
            pl.BlockSpec((1, 256), lambda i, k: (0, 0)),
            pl.BlockSpec((1, 256), lambda i, k: (0, 0)),
        ],
        out_specs=[
            pl.BlockSpec((512, 256), lambda i, k: (i, 0)),
            pl.BlockSpec((1, 512), lambda i, k: (0, i)),
            pl.BlockSpec((1, 512), lambda i, k: (0, i)),
        ],
        out_shape=[
            jax.ShapeDtypeStruct((NPAD, 256), jnp.float32),
            jax.ShapeDtypeStruct((1, NPAD), jnp.float32),
            jax.ShapeDtypeStruct((1, NPAD), jnp.float32),
        ],
    )(outp1, b1r, W2, as2f, ad2f)


def _final_body(p_ref, b_ref, out_ref):
    o0 = p_ref[0, 0] + p_ref[1, 0]
    o1 = p_ref[0, 1] + p_ref[1, 1]
    out_ref[...] = jnp.concatenate([o0, o1], axis=1) + b_ref[...]


def _final(outp2, b2r):
    return pl.pallas_call(
        _final_body,
        grid=(NPAD // 512,),
        in_specs=[
            pl.BlockSpec((2, 2, 512, 128), lambda i: (0, 0, i, 0)),
            pl.BlockSpec((1, 256), lambda i: (0, 0)),
        ],
        out_specs=pl.BlockSpec((512, 256), lambda i: (i, 0)),
        out_shape=jax.ShapeDtypeStruct((NPAD, 256), jnp.float32),
    )(outp2, b2r)


def _sc_mesh():
    return plsc.VectorSubcoreMesh(
        core_axis_name="c", subcore_axis_name="s", num_cores=NC, num_subcores=NS
    )


def _make_coef_kernel(H):
    @functools.partial(
        pl.kernel,
        out_type=jax.ShapeDtypeStruct((H, EPAD), jnp.float32),
        mesh=_sc_mesh(),
        compiler_params=pltpu.CompilerParams(needs_layout_passes=False),
        scratch_types=[
            pltpu.VMEM((DCHUNK,), jnp.int32),
            pltpu.VMEM((DCHUNK,), jnp.int32),
            pltpu.VMEM((TCHUNK,), jnp.int32),
            pltpu.VMEM((TCHUNK,), jnp.int32),
            pltpu.VMEM((NPAD,), jnp.float32),
            pltpu.VMEM((NPAD,), jnp.float32),
            pltpu.VMEM((128, 128), jnp.float32),
            pltpu.VMEM((128, 128), jnp.float32),
            pltpu.VMEM((TCHUNK,), jnp.float32),
            pltpu.VMEM((1, 128), jnp.int32),
            pltpu.VMEM_SHARED((128, 128), jnp.float32),
        ],
    )
    def coef_kernel(src_hbm, dst_hbm, as_hbm, ad_hbm, z_hbm, idx_hbm, coef_hbm,
                    src_d, dst_d, src_c, dst_c, as_t, ad_t, dacc, den_t,
                    coef_buf, idxv, denom_sh):
        c = lax.axis_index("c")
        s = lax.axis_index("s")
        doff = pl.multiple_of(s * DCHUNK, 8)
        woff = pl.multiple_of((c * NS + s) * TCHUNK, 8)
        pltpu.sync_copy(src_hbm.at[pl.ds(doff, DCHUNK)], src_d)
        pltpu.sync_copy(dst_hbm.at[pl.ds(doff, DCHUNK)], dst_d)
        pltpu.sync_copy(src_hbm.at[pl.ds(woff, TCHUNK)], src_c)
        pltpu.sync_copy(dst_hbm.at[pl.ds(woff, TCHUNK)], dst_c)
        pltpu.sync_copy(idx_hbm, idxv)

        for h in range(H):
            pltpu.sync_copy(as_hbm.at[h], as_t)
            pltpu.sync_copy(ad_hbm.at[h], ad_t)
            pltpu.sync_copy(z_hbm, dacc)

            @pl.when(s == 0)
            def _():
                pltpu.sync_copy(z_hbm, denom_sh)

            plsc.subcore_barrier()

            @pl.loop(0, DCHUNK // 16)
            def _(e):
                sl = pl.ds(e * 16, 16)
                sv = src_d[sl]
                dv = dst_d[sl]
                a = plsc.load_gather(as_t, [sv]) + plsc.load_gather(ad_t, [dv])
                a = jnp.where(a >= 0.0, a, 0.2 * a)
                plsc.addupdate_scatter(dacc, [dv >> 7, dv & 127], jnp.exp(a))

            pltpu.sync_copy(dacc, denom_sh.at[idxv.at[0]], add=True)
            plsc.subcore_barrier()
            pltpu.sync_copy(denom_sh, den_t)

            @pl.loop(0, TCHUNK // 16)
            def _(e):
                sl = pl.ds(e * 16, 16)
                sv = src_c[sl]
                dv = dst_c[sl]
                a = plsc.load_gather(as_t, [sv]) + plsc.load_gather(ad_t, [dv])
                a = jnp.where(a >= 0.0, a, 0.2 * a)
                dn = plsc.load_gather(den_t, [dv >> 7, dv & 127])
                coef_buf[sl] = jnp.exp(a) / (dn + 1e-16)

            pltpu.sync_copy(coef_buf, coef_hbm.at[h, pl.ds(woff, TCHUNK)])
            plsc.subcore_barrier()

    return coef_kernel


def _make_agg_kernel(nslab):
    @functools.partial(
        pl.kernel,
        out_type=jax.ShapeDtypeStruct((NC, nslab, NPAD, 128), jnp.float32),
        mesh=_sc_mesh(),
        compiler_params=pltpu.CompilerParams(needs_layout_passes=False),
        scratch_types=[
            pltpu.VMEM((TBLK, 128), jnp.int32),
            pltpu.VMEM((TBLK, 128), jnp.int32),
            pltpu.VMEM((128, 128), jnp.float32),
            pltpu.VMEM((128, 128), jnp.float32),
            pltpu.VMEM((8, 128), jnp.float32),
            pltpu.VMEM_SHARED((NPAD, 128), jnp.float32),
            pltpu.SemaphoreType.DMA,
            pltpu.SemaphoreType.DMA,
            pltpu.SemaphoreType.DMA,
            pltpu.SemaphoreType.DMA,
        ],
    )
    def agg_kernel(src2_hbm, dst2_hbm, coef_hbm, xps_hbm, zrow_hbm, outp_hbm,
                   src2, dst2, rows0, rows1, cbuf, acc,
                   gsem0, gsem1, ssem0, ssem1):
        c = lax.axis_index("c")
        s = lax.axis_index("s")
        wid = c * NS + s
        stripe = pl.multiple_of(s * NSTRIPE, 8)
        pltpu.sync_copy(src2_hbm.at[wid], src2)
        pltpu.sync_copy(dst2_hbm.at[wid], dst2)

        rbufs = (rows0, rows1)
        gsems = (gsem0, gsem1)
        ssems = (ssem0, ssem1)

        def _scale(rows_ref, crow):
            @pl.loop(0, 8)
            def _(g):
                cv = cbuf[crow, pl.ds(g * 16, 16)]
                for e in range(16):
                    sc = cv[e]
                    erow = g * 16 + e
                    for r in range(8):
                        rows_ref[erow, pl.ds(r * 16, 16)] *= sc

        @pl.loop(0, nslab)
        def _(slab):
            head = slab >> 1
            xs = xps_hbm.at[slab]

            @pl.loop(0, NSTRIPE // 128)
            def _(j):
                pltpu.sync_copy(zrow_hbm, acc.at[pl.ds(stripe + j * 128, 128)])

            plsc.subcore_barrier()

            for b in range(2):
                pltpu.async_copy(xs.at[src2.at[b]], rbufs[b], gsems[b])
                pltpu.sync_copy(coef_hbm.at[head, wid, b], cbuf.at[b])

            @pl.loop(0, TBLK // 2)
            def _(pair):
                j0 = 2 * pair
                for b in range(2):
                    jb = j0 + b
                    pltpu.make_async_copy(
                        xs.at[src2.at[jb]], rbufs[b], gsems[b]).wait()
                    _scale(rbufs[b], b)
                    pltpu.async_copy(
                        rbufs[b], acc.at[dst2.at[jb]], ssems[b], add=True)

                @pl.when(pair < TBLK // 2 - 1)
                def _():
                    for b in range(2):
                        jb = j0 + b
                        pltpu.make_async_copy(
                            rbufs[b], acc.at[dst2.at[jb]], ssems[b]).wait()
                        pltpu.sync_copy(
                            coef_hbm.at[head, wid, jb + 2], cbuf.at[b])
                        pltpu.async_copy(
                            xs.at[src2.at[jb + 2]], rbufs[b], gsems[b])

            for b in range(2):
                pltpu.make_async_copy(
                    rbufs[b], acc.at[dst2.at[TBLK - 2 + b]], ssems[b]).wait()

            plsc.subcore_barrier()

            @pl.loop(0, NSTRIPE // 128)
            def _(j):
                roff = pl.multiple_of(stripe + j * 128, 8)
                pltpu.sync_copy(acc.at[pl.ds(roff, 128)], rows0)
                pltpu.sync_copy(rows0, outp_hbm.at[c, slab, pl.ds(roff, 128)])

    return agg_kernel


_coef8 = _make_coef_kernel(8)
_coef1 = _make_coef_kernel(1)
_agg16 = _make_agg_kernel(16)
_agg2 = _make_agg_kernel(2)


def kernel(x, edge_index, W1, as1, ad1, b1, W2, as2, ad2, b2):
    loop = jnp.arange(N, dtype=edge_index.dtype)
    ei = jnp.concatenate([edge_index, jnp.stack([loop, loop])], axis=1)
    padlen = EPAD - ETOT
    src_p = jnp.concatenate(
        [ei[0], jnp.zeros((padlen,), jnp.int32)]).astype(jnp.int32)
    dst_p = jnp.concatenate(
        [ei[1], jnp.full((padlen,), N, jnp.int32)]).astype(jnp.int32)
    src2d = src_p.reshape(NC * NS, TBLK, 128)
    dst2d = dst_p.reshape(NC * NS, TBLK, 128)

    xpad = jnp.pad(x, ((0, NPAD - N), (0, 0)))
    zrow = jnp.zeros((128, 128), jnp.float32)
    idx128 = jnp.arange(128, dtype=jnp.int32).reshape(1, 128)

    xps1, asrc1, adst1 = _project_l1(
        xpad, W1, as1.reshape(1, -1), ad1.reshape(1, -1))
    coef1 = _coef8(src_p, dst_p, asrc1.reshape(8, NPAD),
                   adst1.reshape(8, NPAD), zrow, idx128)
    outp1 = _agg16(src2d, dst2d,
                   coef1.reshape(8, NC * NS, TBLK, 128), xps1, zrow)

    xp2, asrc2, adst2 = _mid(outp1, b1.reshape(16, 1, 128), W2,
                             as2.reshape(1, -1), ad2.reshape(1, -1))
    xps2 = jnp.stack([xp2[:, :128], xp2[:, 128:]], axis=0)
    coef2 = _coef1(src_p, dst_p, asrc2, adst2, zrow, idx128)
    outp2 = _agg2(src2d, dst2d,
                  coef2.reshape(1, NC * NS, TBLK, 128), xps2, zrow)

    out = _final(outp2, b2.reshape(1, -1))
    return out[:N]
